# baseline (speedup 1.0000x reference)
import functools

import jax
import jax.numpy as jnp
from jax import lax
from jax.experimental import pallas as pl
from jax.experimental.pallas import tpu as pltpu
from jax.experimental.pallas import tpu_sc as plsc

N_ADDR = 1000000
D = 64
TEMPERATURE = 0.1
EPS = 1e-8

NW = 32
CHUNK = 400
NCHUNK = N_ADDR // CHUNK
BASE_TRIPS = NCHUNK // NW
EXTRA_W = NCHUNK - BASE_TRIPS * NW


def _hsum(x):
    for sh in (8, 4, 2, 1):
        idx = lax.rem(lax.iota(jnp.int32, 16) + sh, jnp.full((16,), 16, jnp.int32))
        rot = lax.gather(
            x, idx[:, None],
            lax.GatherDimensionNumbers(
                offset_dims=(), collapsed_slice_dims=(0,),
                start_index_map=(0,)),
            slice_sizes=(1,),
            mode=lax.GatherScatterMode.PROMISE_IN_BOUNDS)
        x = x + rot
    return x


def _rsqrt(x):
    xi = lax.bitcast_convert_type(x, jnp.int32)
    yi = jnp.int32(0x5F3759DF) - lax.shift_right_arithmetic(xi, 1)
    y = lax.bitcast_convert_type(yi, jnp.float32)
    for _ in range(3):
        y = y * (1.5 - 0.5 * x * y * y)
    return y


def _sc_body(q_hbm, a_hbm, outv_hbm, outs_hbm, qbuf, buf, vbuf, sbuf):
    wid = lax.axis_index("s") * 2 + lax.axis_index("c")
    trips = BASE_TRIPS + jnp.where(wid < EXTRA_W, 1, 0)

    pltpu.sync_copy(q_hbm.at[0], qbuf)
    q0 = qbuf[pl.ds(0, 16)]
    q1 = qbuf[pl.ds(16, 16)]
    q2 = qbuf[pl.ds(32, 16)]
    q3 = qbuf[pl.ds(48, 16)]
    qss = _hsum(q0 * q0 + q1 * q1 + q2 * q2 + q3 * q3)

    def chunk_body(t, carry):
        va0, va1, va2, va3, sacc = carry
        c = wid + NW * t
        pltpu.sync_copy(a_hbm.at[pl.ds(c * CHUNK, CHUNK), :], buf)

        def row_body(r, rc):
            va0, va1, va2, va3, sacc = rc
            a0 = buf[r, pl.ds(0, 16)]
            a1 = buf[r, pl.ds(16, 16)]
            a2 = buf[r, pl.ds(32, 16)]
            a3 = buf[r, pl.ds(48, 16)]
            dot = _hsum(a0 * q0 + a1 * q1 + a2 * q2 + a3 * q3)
            ssq = _hsum(a0 * a0 + a1 * a1 + a2 * a2 + a3 * a3)
            x = ssq * qss
            nrm = x * _rsqrt(x)
            sim = dot / jnp.maximum(nrm, EPS)
            w = jnp.exp((sim - 1.0) * (1.0 / TEMPERATURE))
            return (va0 + w * a0, va1 + w * a1, va2 + w * a2,
                    va3 + w * a3, sacc + w)

        return lax.fori_loop(0, CHUNK, row_body,
                             (va0, va1, va2, va3, sacc))

    z = jnp.zeros((16,), jnp.float32)
    va0, va1, va2, va3, sacc = lax.fori_loop(
        0, trips, chunk_body, (z, z, z, z, z))

    vbuf[pl.ds(0, 16)] = va0
    vbuf[pl.ds(16, 16)] = va1
    vbuf[pl.ds(32, 16)] = va2
    vbuf[pl.ds(48, 16)] = va3
    sbuf[...] = sacc
    pltpu.sync_copy(vbuf, outv_hbm.at[wid])
    pltpu.sync_copy(sbuf, outs_hbm.at[wid])


@jax.jit
def kernel(query_address, addresses):
    mesh = plsc.VectorSubcoreMesh(core_axis_name="c", subcore_axis_name="s")
    run = functools.partial(
        pl.kernel,
        mesh=mesh,
        out_type=[
            jax.ShapeDtypeStruct((NW, D), jnp.float32),
            jax.ShapeDtypeStruct((NW, 16), jnp.float32),
        ],
        scratch_types=[
            pltpu.VMEM((D,), jnp.float32),
            pltpu.VMEM((CHUNK, D), jnp.float32),
            pltpu.VMEM((D,), jnp.float32),
            pltpu.VMEM((16,), jnp.float32),
        ],
    )(_sc_body)
    outv, outs = run(query_address, addresses)
    wsum = jnp.sum(outv, axis=0)
    ssum = jnp.sum(outs[:, 0])
    return wsum / ssum
